# Initial kernel scaffold; baseline (speedup 1.0000x reference)
import jax
import jax.numpy as jnp
from jax.experimental import pallas as pl


def kernel(x, edge_index, W_l, b_l, W_r):
    raise NotImplementedError("write your pallas kernel here")


# speedup vs baseline: 6.0991x; 6.0991x over previous
import functools

import jax
import jax.numpy as jnp
from jax import lax
from jax.experimental import pallas as pl
from jax.experimental.pallas import tpu as pltpu
from jax.experimental.pallas import tpu_sc as plsc

N = 10000
E = 320000
D = 128
NC = 2
NS = 16
NW = NC * NS
EPW = E // NW
CH = 80
NCHUNK = EPW // CH
ZR = 1000
NZW = N // ZR
WB = 200


def _sc_aggregate(x, src, dst, zrows, zdeg, ones):
    mesh = plsc.VectorSubcoreMesh(core_axis_name="c", subcore_axis_name="s")

    @functools.partial(
        pl.kernel,
        out_type=(
            jax.ShapeDtypeStruct((NC, N, D), jnp.float32),
            jax.ShapeDtypeStruct((NC * N,), jnp.float32),
        ),
        mesh=mesh,
        scratch_types=[
            pltpu.VMEM_SHARED((N, D), jnp.float32),
            pltpu.VMEM_SHARED((N,), jnp.float32),
            pltpu.VMEM((CH,), jnp.int32),
            pltpu.VMEM((CH,), jnp.int32),
            pltpu.VMEM((CH, D), jnp.float32),
            pltpu.VMEM((CH,), jnp.float32),
            pltpu.VMEM((WB, D), jnp.float32),
            pltpu.VMEM((ZR,), jnp.float32),
            pltpu.SemaphoreType.DMA,
        ],
    )
    def k(x_hbm, src_hbm, dst_hbm, zr_hbm, zd_hbm, on_hbm,
          acc_out, deg_out, acc_sh, deg_sh, sidx, didx, rows, ones_v,
          stg, stgd, sem):
        cid = lax.axis_index("c")
        sid = lax.axis_index("s")
        w = cid * NS + sid

        @pl.when(sid < NZW)
        def _():
            pltpu.sync_copy(zr_hbm, stg)
            pltpu.sync_copy(zd_hbm, stgd)
            pltpu.sync_copy(stgd, deg_sh.at[pl.ds(sid * ZR, ZR)])
            for j in range(ZR // WB):
                pltpu.sync_copy(stg, acc_sh.at[pl.ds(sid * ZR + j * WB, WB)])

        pltpu.sync_copy(on_hbm, ones_v)
        plsc.subcore_barrier()

        @pl.loop(0, NCHUNK)
        def _(c):
            base = w * EPW + c * CH
            pltpu.sync_copy(src_hbm.at[pl.ds(base, CH)], sidx)
            pltpu.sync_copy(dst_hbm.at[pl.ds(base, CH)], didx)
            pltpu.async_copy(x_hbm.at[sidx], rows, sem).wait()
            pltpu.sync_copy(rows, acc_sh.at[didx], add=True)
            pltpu.sync_copy(ones_v, deg_sh.at[didx], add=True)

        plsc.subcore_barrier()

        @pl.when(sid < NZW)
        def _():
            pltpu.sync_copy(deg_sh.at[pl.ds(sid * ZR, ZR)], stgd)
            pltpu.sync_copy(stgd, deg_out.at[pl.ds(cid * N + sid * ZR, ZR)])
            for j in range(ZR // WB):
                row0 = sid * ZR + j * WB
                pltpu.sync_copy(acc_sh.at[pl.ds(row0, WB)], stg)
                pltpu.sync_copy(stg, acc_out.at[cid, pl.ds(row0, WB)])

    return k(x, src, dst, zrows, zdeg, ones)


def _tc_update(acc, degT, x, wl_t, wr_t, b):
    R = 1000
    G = N // R

    def body(acc_ref, deg_ref, x_ref, wl_ref, wr_ref, b_ref, o_ref):
        a = acc_ref[0] + acc_ref[1]
        d = deg_ref[:, 0:1] + deg_ref[:, 1:2]
        r = 1.0 / jnp.maximum(d, 1.0)
        aggr = a * r
        out = (jnp.dot(aggr, wl_ref[...], preferred_element_type=jnp.float32)
               + jnp.dot(x_ref[...], wr_ref[...],
                         preferred_element_type=jnp.float32)
               + b_ref[...])
        o_ref[...] = jnp.maximum(out, 0.0)

    return pl.pallas_call(
        body,
        grid=(G,),
        in_specs=[
            pl.BlockSpec((2, R, D), lambda i: (0, i, 0)),
            pl.BlockSpec((R, 2), lambda i: (i, 0)),
            pl.BlockSpec((R, D), lambda i: (i, 0)),
            pl.BlockSpec((D, D), lambda i: (0, 0)),
            pl.BlockSpec((D, D), lambda i: (0, 0)),
            pl.BlockSpec((1, D), lambda i: (0, 0)),
        ],
        out_specs=pl.BlockSpec((R, D), lambda i: (i, 0)),
        out_shape=jax.ShapeDtypeStruct((N, D), jnp.float32),
    )(acc, degT, x, wl_t, wr_t, b)


def kernel(x, edge_index, W_l, b_l, W_r):
    src = edge_index[0]
    dst = edge_index[1]
    zrows = jnp.zeros((WB, D), jnp.float32)
    zdeg = jnp.zeros((ZR,), jnp.float32)
    ones = jnp.ones((CH,), jnp.float32)
    acc, deg = _sc_aggregate(x, src, dst, zrows, zdeg, ones)
    degT = deg.reshape(NC, N).T
    return _tc_update(acc, degT, x, W_l.T, W_r.T, b_l[None, :])
